# baseline (speedup 1.0000x reference)
import functools

import jax
import jax.numpy as jnp
from jax import lax
from jax.experimental import pallas as pl
from jax.experimental.pallas import tpu as pltpu
from jax.experimental.pallas import tpu_sc as plsc

B = 16384
D = 64
NV = 1000000
NC = 2
NS = 16
NW = NC * NS
BPW = B // NW
TPW = 244
NT = 7812
CPW = TPW * 128
CW = 512
NCHUNK = TPW * 128 // CW
MCAP = 1024
NSR = 3
SRW = 10240
SRCAP = 320
RCAP = B + 16

_mesh = plsc.VectorSubcoreMesh(core_axis_name="c", subcore_axis_name="s")
_params = pltpu.CompilerParams(needs_layout_passes=False,
                               disable_bounds_checks=True,
                               )


def _filter_indices(idx_hbm, ibuf, mi, mb, lo, hi, lo2, hi2):
    def chunk_body(c, n):
        pltpu.sync_copy(idx_hbm.at[pl.ds(c * 2048, 2048)], ibuf)

        def group_body(g8, n):
            for u in range(8):
                g = g8 * 8 + u
                iv = ibuf[pl.ds(g * 16, 16)]
                bv = c * 2048 + g * 16 + lax.iota(jnp.int32, 16)
                mask = (((iv >= lo) & (iv < hi))
                        | ((iv >= lo2) & (iv < hi2)))
                plsc.store_compressed(mi.at[pl.ds(n, 16)], iv, mask=mask)
                plsc.store_compressed(mb.at[pl.ds(n, 16)], bv, mask=mask)
                n = n + plsc.all_reduce_population_count(mask)[0]
            return n

        return lax.fori_loop(0, 16, group_body, n)

    return lax.fori_loop(0, B // 2048, chunk_body, jnp.int32(0))


def plsc_drain_one(recbuf, rec_out, bidx, ssem):
    pltpu.make_async_copy(recbuf, rec_out.at[bidx], ssem).wait()


def _bucket(src_i, src_b, n, lo, hi, dst_i, dst_b, off):
    j16 = lax.iota(jnp.int32, 16)

    def body(g, m):
        iv = src_i[pl.ds(g * 16, 16)]
        bv = src_b[pl.ds(g * 16, 16)]
        inb = (j16 + g * 16) < n
        mask = (iv >= lo) & (iv < hi) & inb
        plsc.store_compressed(dst_i.at[pl.ds(off + m, 16)], iv, mask=mask)
        plsc.store_compressed(dst_b.at[pl.ds(off + m, 16)], bv, mask=mask)
        return m + plsc.all_reduce_population_count(mask)[0]

    return lax.fori_loop(0, (n + 15) // 16, body, jnp.int32(0))


def _extract_chunk(chunk_v, src_i, src_b, off, cnt, cc0, cwidth, recbuf,
                   bidx, rec_out, ssem, sbuf_i, sbuf_b):
    j16 = lax.iota(jnp.int32, 16)

    def refilter(g, n):
        iv = src_i[pl.ds(off + g * 16, 16)]
        bv = src_b[pl.ds(off + g * 16, 16)]
        inb = (j16 + g * 16) < cnt
        mask = (iv >= cc0) & (iv < cc0 + cwidth) & inb
        plsc.store_compressed(sbuf_i.at[pl.ds(n, 16)], iv, mask=mask)
        plsc.store_compressed(sbuf_b.at[pl.ds(n, 16)], bv, mask=mask)
        return n + plsc.all_reduce_population_count(mask)[0]

    nc = lax.fori_loop(0, (cnt + 15) // 16, refilter, jnp.int32(0))

    def scatter_group(s, _):
        plsc_drain_one(recbuf, rec_out, bidx, ssem)
        bl = sbuf_b[pl.ds(s * 16, 16)]
        valid = (j16 + s * 16) < nc
        bidx[...] = jnp.where(valid, bl, B + j16)
        iv16 = jnp.where(valid, sbuf_i[pl.ds(s * 16, 16)] - cc0, 0)

        def jblock(q, _):
            for u in range(4):
                jv = jnp.broadcast_to(q * 4 + u, (16,))
                vals = plsc.load_gather(chunk_v, [jv, iv16])
                plsc.store_scatter(recbuf, [j16, jv], vals)
            return 0

        lax.fori_loop(0, D // 4, jblock, 0)
        pltpu.async_copy(recbuf, rec_out.at[bidx], ssem)
        return 0

    lax.fori_loop(0, (nc + 15) // 16, scatter_group, 0)


@functools.partial(
    pl.kernel,
    mesh=_mesh,
    out_type=tuple(jax.ShapeDtypeStruct((RCAP, 128), jnp.float32)
                   for _ in range(4)),
    scratch_types=[
        pltpu.VMEM((D, CW), jnp.float32),
        pltpu.VMEM((D, CW), jnp.float32),
        pltpu.VMEM((2048,), jnp.int32),
        tuple(pltpu.VMEM((MCAP,), jnp.int32) for _ in range(4)),
        tuple(pltpu.VMEM((MCAP,), jnp.int32) for _ in range(4)),
        tuple(pltpu.VMEM((NSR * SRCAP,), jnp.int32) for _ in range(4)),
        tuple(pltpu.VMEM((NSR * SRCAP,), jnp.int32) for _ in range(4)),
        pltpu.VMEM((64,), jnp.int32),
        pltpu.VMEM((64,), jnp.int32),
        tuple(pltpu.VMEM((16, 128), jnp.float32) for _ in range(4)),
        tuple(pltpu.VMEM((16,), jnp.int32) for _ in range(4)),
        pltpu.SemaphoreType.DMA,
        tuple(pltpu.SemaphoreType.DMA for _ in range(4)),
    ],
    compiler_params=_params,
)
def _scan(user_t, item_t, tail_u, tail_i, users_hbm, pos_hbm, pot_hbm,
          neg_hbm, rec_u, rec_p, rec_t, rec_n,
          chunk_a, chunk_b, ibuf, mi, mb, sri, srb, sbuf_i, sbuf_b, recbufs,
          bidxs, gsem, ssems):
    wid = lax.axis_index("s") * NC + lax.axis_index("c")
    c0 = wid * CPW
    last = wid == NW - 1
    first = wid == 0
    lo = c0
    hi = c0 + CPW + jnp.where(last, 512, 0)
    lo2 = jnp.where(first, NT * 128, 1)
    hi2 = jnp.where(first, NV, 0)

    counts = []
    for k, idx_hbm in enumerate((users_hbm, pos_hbm, pot_hbm, neg_hbm)):
        counts.append(_filter_indices(idx_hbm, ibuf, mi[k], mb[k],
                                      lo, hi, lo2, hi2))

    for sl in range(4):
        bidxs[sl][...] = B + lax.iota(jnp.int32, 16)
        pltpu.async_copy(recbufs[sl], rec_u.at[bidxs[sl]], ssems[sl])
    slot_counter = [0]

    for tab, tail, ks in ((user_t, tail_u, (0,)), (item_t, tail_i, (1, 2, 3))):
        recs = (rec_u, rec_p, rec_t, rec_n)

        def extract(buf, k, soff, scnt, cc0, cwidth, src=None):
            si = sri[k] if src is None else src[0]
            sb = srb[k] if src is None else src[1]
            sl = slot_counter[0] % 4
            slot_counter[0] += 1
            _extract_chunk(buf, si, sb, soff, scnt, cc0, cwidth,
                           recbufs[sl], bidxs[sl], recs[k], ssems[sl],
                           sbuf_i, sbuf_b)

        def fire(c, buf):
            return pltpu.async_copy(tab.at[:, pl.ds(c0 + c * CW, CW)],
                                    buf, gsem)

        def drain(buf):
            pltpu.make_async_copy(tab.at[:, pl.ds(c0, CW)], buf, gsem).wait()

        fire(0, chunk_a)
        srcnt = {}
        for r in range(NSR):
            sr_lo = c0 + r * SRW
            sr_hi = c0 + (r * SRW + SRW if r < NSR - 1 else CPW)
            for k in ks:
                srcnt[k] = _bucket(mi[k], mb[k], counts[k], sr_lo, sr_hi,
                                   sri[k], srb[k], r * SRCAP)
            npairs = 10

            def pair_body(i, _, r=r, scn=dict(srcnt)):
                ca = r * 20 + 2 * i
                fire(ca + 1, chunk_b)
                drain(chunk_a)
                for k in ks:
                    extract(chunk_a, k, r * SRCAP, scn[k], c0 + ca * CW, CW)
                fire(ca + 2, chunk_a)
                drain(chunk_b)
                for k in ks:
                    extract(chunk_b, k, r * SRCAP, scn[k],
                            c0 + (ca + 1) * CW, CW)
                return 0

            lax.fori_loop(0, npairs, pair_body, 0)

        drain(chunk_a)
        for k in ks:
            extract(chunk_a, k, (NSR - 1) * SRCAP, srcnt[k],
                    c0 + (NCHUNK - 1) * CW, CW)

        @pl.when(last)
        def _():
            pltpu.async_copy(tab.at[:, pl.ds(NW * CPW, CW)],
                             chunk_a, gsem).wait()
            for k in ks:
                extract(chunk_a, k, 0, counts[k], NW * CPW, CW,
                        src=(mi[k], mb[k]))

        @pl.when(first)
        def _():
            pltpu.sync_copy(tail, chunk_a.at[:, pl.ds(0, 128)])
            for k in ks:
                extract(chunk_a, k, 0, counts[k], NT * 128, 64,
                        src=(mi[k], mb[k]))

    for sl in range(4):
        plsc_drain_one(recbufs[sl], rec_u, bidxs[sl], ssems[sl])


def _assemble_body(rec_ref, out_ref):
    out_ref[...] = rec_ref[:, :D].T


_assemble = pl.pallas_call(
    _assemble_body,
    grid=(B // 512,),
    in_specs=[pl.BlockSpec((512, 128), lambda i: (i, 0))],
    out_specs=pl.BlockSpec((D, 512), lambda i: (0, i)),
    out_shape=jax.ShapeDtypeStruct((D, B), jnp.float32),
)


def _tail(tab):
    pad = jnp.zeros((64, D), jnp.float32)
    return jnp.concatenate([tab[NT * 128:], pad], axis=0).T


def kernel(user_emb, item_emb, users, pos_items, pot_items, neg_items):
    recs = _scan(user_emb.T, item_emb.T, _tail(user_emb), _tail(item_emb),
                 users.astype(jnp.int32), pos_items.astype(jnp.int32),
                 pot_items.astype(jnp.int32), neg_items.astype(jnp.int32))
    return tuple(_assemble(r).T for r in recs)
